# baseline (speedup 1.0000x reference)
import functools

import jax
import jax.numpy as jnp
from jax import lax
from jax.experimental import pallas as pl
from jax.experimental.pallas import tpu as pltpu
from jax.experimental.pallas import tpu_sc as plsc

NC = 2
NS = 16
NW = NC * NS
L = 16


def _sc_mesh():
    return plsc.VectorSubcoreMesh(core_axis_name="c", subcore_axis_name="s",
                                  num_cores=NC, num_subcores=NS)


def _pack_bf16_pairs(x):
    k = x.shape[1] // 2
    lo = lax.bitcast_convert_type(x[:, :k], jnp.uint32)
    hi = lax.bitcast_convert_type(x[:, k:], jnp.uint32)
    lo16 = lax.shift_right_logical(lo + jnp.uint32(0x8000), jnp.uint32(16))
    hi16 = (hi + jnp.uint32(0x8000)) & jnp.uint32(0xFFFF0000)
    return lax.bitcast_convert_type(hi16 | lo16, jnp.int32)


def _unpack_bf16_pairs(p):
    u = lax.bitcast_convert_type(p, jnp.uint32)
    lo = lax.bitcast_convert_type(lax.shift_left(u, jnp.uint32(16)),
                                  jnp.float32)
    hi = lax.bitcast_convert_type(u & jnp.uint32(0xFFFF0000), jnp.float32)
    return jnp.concatenate([lo, hi], axis=1)


def _mm_body(x_ref, w_ref, b_ref, uh_ref, aa_ref, ba_ref, va_ref):
    d = x_ref.shape[1]
    o = (
        jnp.dot(x_ref[...], w_ref[...], preferred_element_type=jnp.float32)
        + b_ref[...]
    )
    uh_ref[...] = o[:, :d]
    aa_ref[...] = _pack_bf16_pairs(o[:, d:2 * d])
    ba_ref[...] = _pack_bf16_pairs(o[:, 2 * d:3 * d])
    va_ref[...] = _pack_bf16_pairs(o[:, 3 * d:])


def _node_matmul(h, w_all, b_all, block_rows):
    n, d = h.shape
    dout = w_all.shape[1]
    dh = d // 2
    grid = (n // block_rows,)
    nd = jax.ShapeDtypeStruct((n, d), jnp.float32)
    nb = jax.ShapeDtypeStruct((n, d // 2), jnp.int32)
    return pl.pallas_call(
        _mm_body,
        grid=grid,
        in_specs=[
            pl.BlockSpec((block_rows, d), lambda i: (i, 0)),
            pl.BlockSpec((d, dout), lambda i: (0, 0)),
            pl.BlockSpec((1, dout), lambda i: (0, 0)),
        ],
        out_specs=[
            pl.BlockSpec((block_rows, d), lambda i: (i, 0)),
            pl.BlockSpec((block_rows, d // 2), lambda i: (i, 0)),
            pl.BlockSpec((block_rows, d // 2), lambda i: (i, 0)),
            pl.BlockSpec((block_rows, d // 2), lambda i: (i, 0)),
        ],
        out_shape=[nd, nb, nb, nb],
    )(h, w_all, b_all)


_NBUF = 3


def _gather_body(aa_hbm, ba_hbm, va_hbm, dst_hbm, src_hbm,
                 ag_hbm, bg_hbm, vg_hbm,
                 idx_d, idx_s, rows_a, rows_b, rows_v, sems, chunk, n_iter):
    wid = lax.axis_index("c") * NS + lax.axis_index("s")
    per_tile = chunk * n_iter
    base0 = wid * per_tile

    def start_idx(i, b):
        pltpu.async_copy(dst_hbm.at[pl.ds(base0 + i * chunk, chunk)],
                         idx_d.at[b], sems.at[0, b])
        pltpu.async_copy(src_hbm.at[pl.ds(base0 + i * chunk, chunk)],
                         idx_s.at[b], sems.at[1, b])

    def wait_idx(b):
        pltpu.make_async_copy(dst_hbm.at[pl.ds(0, chunk)], idx_d.at[b],
                              sems.at[0, b]).wait()
        pltpu.make_async_copy(src_hbm.at[pl.ds(0, chunk)], idx_s.at[b],
                              sems.at[1, b]).wait()

    def start_gather(b):
        pltpu.async_copy(aa_hbm.at[idx_d.at[b]], rows_a.at[b], sems.at[2, b])
        pltpu.async_copy(ba_hbm.at[idx_s.at[b]], rows_b.at[b], sems.at[3, b])
        pltpu.async_copy(va_hbm.at[idx_s.at[b]], rows_v.at[b], sems.at[4, b])

    def wait_gather(b):
        pltpu.make_async_copy(aa_hbm.at[idx_d.at[b]], rows_a.at[b],
                              sems.at[2, b]).wait()
        pltpu.make_async_copy(ba_hbm.at[idx_s.at[b]], rows_b.at[b],
                              sems.at[3, b]).wait()
        pltpu.make_async_copy(va_hbm.at[idx_s.at[b]], rows_v.at[b],
                              sems.at[4, b]).wait()

    def start_wb(i, b):
        pltpu.async_copy(rows_a.at[b],
                         ag_hbm.at[pl.ds(base0 + i * chunk, chunk)],
                         sems.at[5, b])
        pltpu.async_copy(rows_b.at[b],
                         bg_hbm.at[pl.ds(base0 + i * chunk, chunk)],
                         sems.at[6, b])
        pltpu.async_copy(rows_v.at[b],
                         vg_hbm.at[pl.ds(base0 + i * chunk, chunk)],
                         sems.at[7, b])

    def wait_wb(b):
        pltpu.make_async_copy(rows_a.at[b], ag_hbm.at[pl.ds(0, chunk)],
                              sems.at[5, b]).wait()
        pltpu.make_async_copy(rows_b.at[b], bg_hbm.at[pl.ds(0, chunk)],
                              sems.at[6, b]).wait()
        pltpu.make_async_copy(rows_v.at[b], vg_hbm.at[pl.ds(0, chunk)],
                              sems.at[7, b]).wait()

    def step(i, b, has_next, do_wbwait, has_idx4):
        wait_gather(b)
        start_wb(i, b)
        if has_next:
            b1 = (b + 1) % _NBUF
            wait_idx(b1)
            if do_wbwait:
                wait_wb(b1)
            start_gather(b1)
        if has_idx4:
            start_idx(i + _NBUF, b)

    for i in range(min(_NBUF, n_iter)):
        start_idx(i, i % _NBUF)
    wait_idx(0)
    start_gather(0)

    for i in range(min(_NBUF, n_iter)):
        step(i, i % _NBUF, has_next=i + 1 < n_iter,
             do_wbwait=i >= _NBUF - 1, has_idx4=i + _NBUF < n_iter)

    lo = _NBUF
    hi = max(((n_iter - 1 - _NBUF) // _NBUF) * _NBUF, lo)
    if hi > lo:
        def grp(m, _):
            i0 = lo + m * _NBUF
            for k in range(_NBUF):
                step(i0 + k, k, has_next=True, do_wbwait=True, has_idx4=True)
            return 0

        lax.fori_loop(0, (hi - lo) // _NBUF, grp, 0)

    for i in range(hi, n_iter):
        if i < _NBUF:
            continue
        step(i, i % _NBUF, has_next=i + 1 < n_iter, do_wbwait=True,
             has_idx4=i + _NBUF < n_iter)

    for i in range(max(n_iter - _NBUF, 0), n_iter):
        wait_wb(i % _NBUF)


def _sc_gather(aa, ba, va, dst, src):
    n, d = aa.shape
    e = dst.shape[0]
    chunk = 40
    n_iter = e // (NW * chunk)
    assert e == NW * chunk * n_iter
    k = functools.partial(
        pl.kernel,
        out_type=(jax.ShapeDtypeStruct((e, d), jnp.int32),
                  jax.ShapeDtypeStruct((e, d), jnp.int32),
                  jax.ShapeDtypeStruct((e, d), jnp.int32)),
        mesh=_sc_mesh(),
        scratch_types=[
            pltpu.VMEM((_NBUF, chunk), jnp.int32),
            pltpu.VMEM((_NBUF, chunk), jnp.int32),
            pltpu.VMEM((_NBUF, chunk, d), jnp.int32),
            pltpu.VMEM((_NBUF, chunk, d), jnp.int32),
            pltpu.VMEM((_NBUF, chunk, d), jnp.int32),
            pltpu.SemaphoreType.DMA((8, _NBUF)),
        ],
    )(functools.partial(_gather_body, chunk=chunk, n_iter=n_iter))
    return k(aa, ba, va, dst, src)


def _edge_body(ag_ref, bg_ref, vg_ref, ea_ref, cw_ref, cb_ref, ge_ref,
               be_ref, *rest):
    if len(rest) == 3:
        _, eout_ref, msg_ref = rest
    else:
        eout_ref, msg_ref = rest
    d = 2 * ag_ref.shape[1]
    dh = d // 2
    ce = (
        jnp.dot(ea_ref[...], cw_ref[...], preferred_element_type=jnp.float32)
        + cb_ref[...]
    )
    e = _unpack_bf16_pairs(ag_ref[...]) + _unpack_bf16_pairs(bg_ref[...]) + ce
    gates = jax.nn.sigmoid(e)
    msg = gates * _unpack_bf16_pairs(vg_ref[...])
    msg_ref[0] = msg[:, :dh]
    msg_ref[1] = msg[:, dh:]
    se = e * gates
    mu = jnp.mean(se, axis=-1, keepdims=True)
    var = jnp.mean((se - mu) ** 2, axis=-1, keepdims=True)
    eout_ref[...] = (se - mu) * lax.rsqrt(var + 1e-5) * ge_ref[...] + be_ref[...]


def _edge_math(ag, bg, vg, edge_attr, c_w, c_b, gamma_e, beta_e,
               block_rows, e_buf, e_total, row_off):
    es, dp = ag.shape
    d = 2 * dp
    ed = edge_attr.shape[1]
    dh = d // 2
    grid = (es // block_rows,)
    ob = row_off // block_rows
    in_specs = [
        pl.BlockSpec((block_rows, dp), lambda i: (i, 0)),
        pl.BlockSpec((block_rows, dp), lambda i: (i, 0)),
        pl.BlockSpec((block_rows, dp), lambda i: (i, 0)),
        pl.BlockSpec((block_rows, ed), lambda i: (i, 0)),
        pl.BlockSpec((ed, d), lambda i: (0, 0)),
        pl.BlockSpec((1, d), lambda i: (0, 0)),
        pl.BlockSpec((1, d), lambda i: (0, 0)),
        pl.BlockSpec((1, d), lambda i: (0, 0)),
    ]
    args = [ag, bg, vg, edge_attr, c_w, c_b, gamma_e, beta_e]
    aliases = {}
    if e_buf is not None:
        in_specs.append(pl.BlockSpec(memory_space=pl.ANY))
        args.append(e_buf)
        aliases = {8: 0}
    return pl.pallas_call(
        _edge_body,
        grid=grid,
        in_specs=in_specs,
        out_specs=[
            pl.BlockSpec((block_rows, d), lambda i: (ob + i, 0)),
            pl.BlockSpec((2, block_rows, dh), lambda i: (0, i, 0)),
        ],
        out_shape=[
            jax.ShapeDtypeStruct((e_total, d), jnp.float32),
            jax.ShapeDtypeStruct((2, es, dh), jnp.float32),
        ],
        input_output_aliases=aliases,
    )(*args)


def _scatter_body(msg_hbm, dst_hbm, zero_hbm, agg_hbm,
                  idx_d, msg, agg_sp, sems, chunk, n_iter, dh, nbuf):
    c = lax.axis_index("c")
    s = lax.axis_index("s")
    n = agg_sp.shape[0]
    e2 = msg_hbm.shape[0]
    e = e2 // 2
    per_tile = chunk * n_iter
    base0 = s * per_tile
    ce0 = c * e

    def start_loads(i, b):
        pltpu.async_copy(dst_hbm.at[pl.ds(base0 + i * chunk, chunk)],
                         idx_d.at[b], sems.at[0, b])
        pltpu.async_copy(msg_hbm.at[pl.ds(ce0 + base0 + i * chunk, chunk)],
                         msg.at[b], sems.at[1, b])

    def wait_loads(b):
        pltpu.make_async_copy(dst_hbm.at[pl.ds(0, chunk)], idx_d.at[b],
                              sems.at[0, b]).wait()
        pltpu.make_async_copy(msg_hbm.at[pl.ds(0, chunk)], msg.at[b],
                              sems.at[1, b]).wait()

    def start_scadd(b):
        pltpu.async_copy(msg.at[b], agg_sp.at[idx_d.at[b]], sems.at[2, b],
                         add=True)

    def wait_scadd(b):
        pltpu.make_async_copy(msg.at[b], agg_sp.at[idx_d.at[b]],
                              sems.at[2, b]).wait()

    def step(i, b, has_idxn):
        wait_loads(b)
        start_scadd(b)
        if has_idxn:
            wait_scadd(b)
            start_loads(i + nbuf, b)

    for i in range(min(nbuf, n_iter)):
        start_loads(i, i % nbuf)

    rpt = -(-n // NS) // 8 * 8
    lastr = n - (NS - 1) * rpt

    @pl.when(s < NS - 1)
    def _():
        pltpu.sync_copy(zero_hbm.at[pl.ds(s * rpt, rpt)],
                        agg_sp.at[pl.ds(s * rpt, rpt)])

    @pl.when(s == NS - 1)
    def _():
        pltpu.sync_copy(zero_hbm.at[pl.ds((NS - 1) * rpt, lastr)],
                        agg_sp.at[pl.ds((NS - 1) * rpt, lastr)])

    plsc.subcore_barrier()

    for i in range(min(nbuf, n_iter)):
        step(i, i % nbuf, has_idxn=i + nbuf < n_iter)

    lo = nbuf
    hi = max(((n_iter - 1 - nbuf) // nbuf) * nbuf, lo)
    if hi > lo:
        def grp(m, _):
            i0 = lo + m * nbuf
            for k in range(nbuf):
                step(i0 + k, k, has_idxn=True)
            return 0

        lax.fori_loop(0, (hi - lo) // nbuf, grp, 0)

    for i in range(hi, n_iter):
        if i < nbuf:
            continue
        step(i, i % nbuf, has_idxn=i + nbuf < n_iter)

    for i in range(max(n_iter - nbuf, 0), n_iter):
        wait_scadd(i % nbuf)

    plsc.subcore_barrier()

    @pl.when(s < NS - 1)
    def _():
        pltpu.sync_copy(agg_sp.at[pl.ds(s * rpt, rpt)],
                        agg_hbm.at[pl.ds(c * n + s * rpt, rpt)])

    @pl.when(s == NS - 1)
    def _():
        pltpu.sync_copy(agg_sp.at[pl.ds((NS - 1) * rpt, lastr)],
                        agg_hbm.at[pl.ds(c * n + (NS - 1) * rpt, lastr)])


def _sc_scatter(msg2, dst, zeros_half):
    e2, dh = msg2.shape
    e = e2 // 2
    n = zeros_half.shape[0]
    chunk = 80
    nbuf = 3
    n_iter = e // (NS * chunk)
    assert e == NS * chunk * n_iter
    k = functools.partial(
        pl.kernel,
        out_type=jax.ShapeDtypeStruct((2 * n, dh), jnp.float32),
        mesh=_sc_mesh(),
        scratch_types=[
            pltpu.VMEM((nbuf, chunk), jnp.int32),
            pltpu.VMEM((nbuf, chunk, dh), jnp.float32),
            pltpu.VMEM_SHARED((n, dh), jnp.float32),
            pltpu.SemaphoreType.DMA((3, nbuf)),
        ],
    )(functools.partial(_scatter_body, chunk=chunk, n_iter=n_iter, dh=dh,
                        nbuf=nbuf))
    return k(msg2, dst, zeros_half)


def _epi_body(h_ref, uh_ref, gh_ref, bh_ref, *rest):
    agg_refs, o_ref = rest[:-1], rest[-1]
    agg = sum(jnp.concatenate([a[0], a[1]], axis=1) for a in agg_refs)
    x = uh_ref[...] + agg
    out = x * jax.nn.sigmoid(x)
    y = h_ref[...] + out
    mu = jnp.mean(y, axis=-1, keepdims=True)
    var = jnp.mean((y - mu) ** 2, axis=-1, keepdims=True)
    o_ref[...] = (y - mu) * lax.rsqrt(var + 1e-5) * gh_ref[...] + bh_ref[...]


def _epilogue(h, uh, agg2s, gamma_h, beta_h, block_rows):
    n, d = h.shape
    dh = d // 2
    grid = (n // block_rows,)
    return pl.pallas_call(
        _epi_body,
        grid=grid,
        in_specs=[
            pl.BlockSpec((block_rows, d), lambda i: (i, 0)),
            pl.BlockSpec((block_rows, d), lambda i: (i, 0)),
            pl.BlockSpec((1, d), lambda i: (0, 0)),
            pl.BlockSpec((1, d), lambda i: (0, 0)),
        ] + [
            pl.BlockSpec((2, block_rows, dh), lambda i: (0, i, 0))
            for _ in agg2s
        ],
        out_specs=pl.BlockSpec((block_rows, d), lambda i: (i, 0)),
        out_shape=jax.ShapeDtypeStruct((n, d), jnp.float32),
    )(h, uh, gamma_h, beta_h, *agg2s)


def kernel(h, edge_index, edge_attr, U_w, U_b, V_w, V_b, A_w, A_b, B_w, B_b,
           C_w, C_b, gamma_h, beta_h, gamma_e, beta_e):
    n, d = h.shape
    dh = d // 2
    src = edge_index[0]
    dst = edge_index[1]

    w_all = jnp.concatenate([U_w, A_w, B_w, V_w], axis=1)
    b_all = jnp.concatenate([U_b, A_b, B_b, V_b])[None, :]
    uh, aa, ba, va2 = _node_matmul(h, w_all, b_all, block_rows=2000)

    e = src.shape[0]
    unit = NW * 40
    n_units = e // unit
    assert e == unit * n_units
    nslab = 4
    per = n_units // nslab
    sizes = [(per + (1 if i < n_units % nslab else 0)) * unit
             for i in range(nslab)]
    starts = [sum(sizes[:i]) for i in range(nslab)]

    zeros_half = jnp.zeros((n, dh), jnp.float32)

    e_buf = jnp.zeros((0,))
    agg2s = []
    e_total = e
    for si, (st, sz) in enumerate(zip(starts, sizes)):
        dst_s = lax.slice(dst, (st,), (st + sz,))
        src_s = lax.slice(src, (st,), (st + sz,))
        ea_s = lax.slice(edge_attr, (st, 0), (st + sz, edge_attr.shape[1]))
        ag, bg, vg = _sc_gather(aa, ba, va2, dst_s, src_s)
        e_init = None if si == 0 else e_buf
        e_buf, msg2 = _edge_math(ag, bg, vg, ea_s, C_w, C_b[None, :],
                                 gamma_e[None, :], beta_e[None, :],
                                 block_rows=1280, e_buf=e_init,
                                 e_total=e_total, row_off=st)
        agg2s.append(_sc_scatter(msg2.reshape(-1, dh), dst_s, zeros_half))

    e_out = e_buf
    h_out = _epilogue(h, uh, [a.reshape(2, n, dh) for a in agg2s],
                      gamma_h[None, :], beta_h[None, :], block_rows=1000)
    return (h_out, e_out)


# speedup vs baseline: 1.0207x; 1.0207x over previous
import functools

import jax
import jax.numpy as jnp
from jax import lax
from jax.experimental import pallas as pl
from jax.experimental.pallas import tpu as pltpu
from jax.experimental.pallas import tpu_sc as plsc

NC = 2
NS = 16
NW = NC * NS
L = 16


def _sc_mesh():
    return plsc.VectorSubcoreMesh(core_axis_name="c", subcore_axis_name="s",
                                  num_cores=NC, num_subcores=NS)


def _pack_bf16_pairs(x):
    k = x.shape[1] // 2
    lo = lax.bitcast_convert_type(x[:, :k], jnp.uint32)
    hi = lax.bitcast_convert_type(x[:, k:], jnp.uint32)
    lo16 = lax.shift_right_logical(lo + jnp.uint32(0x8000), jnp.uint32(16))
    hi16 = (hi + jnp.uint32(0x8000)) & jnp.uint32(0xFFFF0000)
    return lax.bitcast_convert_type(hi16 | lo16, jnp.int32)


def _unpack_bf16_pairs(p):
    u = lax.bitcast_convert_type(p, jnp.uint32)
    lo = lax.bitcast_convert_type(lax.shift_left(u, jnp.uint32(16)),
                                  jnp.float32)
    hi = lax.bitcast_convert_type(u & jnp.uint32(0xFFFF0000), jnp.float32)
    return jnp.concatenate([lo, hi], axis=1)


def _mm_body(x_ref, w_ref, b_ref, uh_ref, aa_ref, ba_ref, va_ref):
    d = x_ref.shape[1]
    o = (
        jnp.dot(x_ref[...], w_ref[...], preferred_element_type=jnp.float32)
        + b_ref[...]
    )
    uh_ref[...] = o[:, :d]
    aa_ref[...] = _pack_bf16_pairs(o[:, d:2 * d])
    ba_ref[...] = _pack_bf16_pairs(o[:, 2 * d:3 * d])
    va_ref[...] = _pack_bf16_pairs(o[:, 3 * d:])


def _node_matmul(h, w_all, b_all, block_rows):
    n, d = h.shape
    dout = w_all.shape[1]
    dh = d // 2
    grid = (n // block_rows,)
    nd = jax.ShapeDtypeStruct((n, d), jnp.float32)
    nb = jax.ShapeDtypeStruct((n, d // 2), jnp.int32)
    return pl.pallas_call(
        _mm_body,
        grid=grid,
        in_specs=[
            pl.BlockSpec((block_rows, d), lambda i: (i, 0)),
            pl.BlockSpec((d, dout), lambda i: (0, 0)),
            pl.BlockSpec((1, dout), lambda i: (0, 0)),
        ],
        out_specs=[
            pl.BlockSpec((block_rows, d), lambda i: (i, 0)),
            pl.BlockSpec((block_rows, d // 2), lambda i: (i, 0)),
            pl.BlockSpec((block_rows, d // 2), lambda i: (i, 0)),
            pl.BlockSpec((block_rows, d // 2), lambda i: (i, 0)),
        ],
        out_shape=[nd, nb, nb, nb],
    )(h, w_all, b_all)


_NBUF = 3


def _gather_body(aa_hbm, ba_hbm, va_hbm, dst_hbm, src_hbm,
                 ag_hbm, bg_hbm, vg_hbm,
                 idx_d, idx_s, rows_a, rows_b, rows_v, sems, chunk, n_iter):
    wid = lax.axis_index("c") * NS + lax.axis_index("s")
    per_tile = chunk * n_iter
    base0 = wid * per_tile

    def start_idx(i, b):
        pltpu.async_copy(dst_hbm.at[pl.ds(base0 + i * chunk, chunk)],
                         idx_d.at[b], sems.at[0, b])
        pltpu.async_copy(src_hbm.at[pl.ds(base0 + i * chunk, chunk)],
                         idx_s.at[b], sems.at[1, b])

    def wait_idx(b):
        pltpu.make_async_copy(dst_hbm.at[pl.ds(0, chunk)], idx_d.at[b],
                              sems.at[0, b]).wait()
        pltpu.make_async_copy(src_hbm.at[pl.ds(0, chunk)], idx_s.at[b],
                              sems.at[1, b]).wait()

    def start_gather(b):
        pltpu.async_copy(aa_hbm.at[idx_d.at[b]], rows_a.at[b], sems.at[2, b])
        pltpu.async_copy(ba_hbm.at[idx_s.at[b]], rows_b.at[b], sems.at[3, b])
        pltpu.async_copy(va_hbm.at[idx_s.at[b]], rows_v.at[b], sems.at[4, b])

    def wait_gather(b):
        pltpu.make_async_copy(aa_hbm.at[idx_d.at[b]], rows_a.at[b],
                              sems.at[2, b]).wait()
        pltpu.make_async_copy(ba_hbm.at[idx_s.at[b]], rows_b.at[b],
                              sems.at[3, b]).wait()
        pltpu.make_async_copy(va_hbm.at[idx_s.at[b]], rows_v.at[b],
                              sems.at[4, b]).wait()

    def start_wb(i, b):
        pltpu.async_copy(rows_a.at[b],
                         ag_hbm.at[pl.ds(base0 + i * chunk, chunk)],
                         sems.at[5, b])
        pltpu.async_copy(rows_b.at[b],
                         bg_hbm.at[pl.ds(base0 + i * chunk, chunk)],
                         sems.at[6, b])
        pltpu.async_copy(rows_v.at[b],
                         vg_hbm.at[pl.ds(base0 + i * chunk, chunk)],
                         sems.at[7, b])

    def wait_wb(b):
        pltpu.make_async_copy(rows_a.at[b], ag_hbm.at[pl.ds(0, chunk)],
                              sems.at[5, b]).wait()
        pltpu.make_async_copy(rows_b.at[b], bg_hbm.at[pl.ds(0, chunk)],
                              sems.at[6, b]).wait()
        pltpu.make_async_copy(rows_v.at[b], vg_hbm.at[pl.ds(0, chunk)],
                              sems.at[7, b]).wait()

    def step(i, b, has_next, do_wbwait, has_idx4):
        wait_gather(b)
        start_wb(i, b)
        if has_next:
            b1 = (b + 1) % _NBUF
            wait_idx(b1)
            if do_wbwait:
                wait_wb(b1)
            start_gather(b1)
        if has_idx4:
            start_idx(i + _NBUF, b)

    for i in range(min(_NBUF, n_iter)):
        start_idx(i, i % _NBUF)
    wait_idx(0)
    start_gather(0)

    for i in range(min(_NBUF, n_iter)):
        step(i, i % _NBUF, has_next=i + 1 < n_iter,
             do_wbwait=i >= _NBUF - 1, has_idx4=i + _NBUF < n_iter)

    lo = _NBUF
    hi = max(((n_iter - 1 - _NBUF) // _NBUF) * _NBUF, lo)
    if hi > lo:
        def grp(m, _):
            i0 = lo + m * _NBUF
            for k in range(_NBUF):
                step(i0 + k, k, has_next=True, do_wbwait=True, has_idx4=True)
            return 0

        lax.fori_loop(0, (hi - lo) // _NBUF, grp, 0)

    for i in range(hi, n_iter):
        if i < _NBUF:
            continue
        step(i, i % _NBUF, has_next=i + 1 < n_iter, do_wbwait=True,
             has_idx4=i + _NBUF < n_iter)

    for i in range(max(n_iter - _NBUF, 0), n_iter):
        wait_wb(i % _NBUF)


def _sc_gather(aa, ba, va, dst, src):
    n, d = aa.shape
    e = dst.shape[0]
    chunk = 40
    n_iter = e // (NW * chunk)
    assert e == NW * chunk * n_iter
    k = functools.partial(
        pl.kernel,
        out_type=(jax.ShapeDtypeStruct((e, d), jnp.int32),
                  jax.ShapeDtypeStruct((e, d), jnp.int32),
                  jax.ShapeDtypeStruct((e, d), jnp.int32)),
        mesh=_sc_mesh(),
        scratch_types=[
            pltpu.VMEM((_NBUF, chunk), jnp.int32),
            pltpu.VMEM((_NBUF, chunk), jnp.int32),
            pltpu.VMEM((_NBUF, chunk, d), jnp.int32),
            pltpu.VMEM((_NBUF, chunk, d), jnp.int32),
            pltpu.VMEM((_NBUF, chunk, d), jnp.int32),
            pltpu.SemaphoreType.DMA((8, _NBUF)),
        ],
    )(functools.partial(_gather_body, chunk=chunk, n_iter=n_iter))
    return k(aa, ba, va, dst, src)


def _edge_body(ag_ref, bg_ref, vg_ref, ea_ref, cw_ref, cb_ref, ge_ref,
               be_ref, *rest):
    if len(rest) == 3:
        _, eout_ref, msg_ref = rest
    else:
        eout_ref, msg_ref = rest
    d = 2 * ag_ref.shape[1]
    dh = d // 2
    ce = (
        jnp.dot(ea_ref[...], cw_ref[...], preferred_element_type=jnp.float32)
        + cb_ref[...]
    )
    e = _unpack_bf16_pairs(ag_ref[...]) + _unpack_bf16_pairs(bg_ref[...]) + ce
    gates = jax.nn.sigmoid(e)
    msg = gates * _unpack_bf16_pairs(vg_ref[...])
    msg_ref[0] = msg[:, :dh]
    msg_ref[1] = msg[:, dh:]
    se = e * gates
    mu = jnp.mean(se, axis=-1, keepdims=True)
    var = jnp.mean((se - mu) ** 2, axis=-1, keepdims=True)
    eout_ref[...] = (se - mu) * lax.rsqrt(var + 1e-5) * ge_ref[...] + be_ref[...]


def _edge_math(ag, bg, vg, edge_attr, c_w, c_b, gamma_e, beta_e,
               block_rows, e_buf, e_total, row_off):
    es, dp = ag.shape
    d = 2 * dp
    ed = edge_attr.shape[1]
    dh = d // 2
    grid = (es // block_rows,)
    ob = row_off // block_rows
    in_specs = [
        pl.BlockSpec((block_rows, dp), lambda i: (i, 0)),
        pl.BlockSpec((block_rows, dp), lambda i: (i, 0)),
        pl.BlockSpec((block_rows, dp), lambda i: (i, 0)),
        pl.BlockSpec((block_rows, ed), lambda i: (i, 0)),
        pl.BlockSpec((ed, d), lambda i: (0, 0)),
        pl.BlockSpec((1, d), lambda i: (0, 0)),
        pl.BlockSpec((1, d), lambda i: (0, 0)),
        pl.BlockSpec((1, d), lambda i: (0, 0)),
    ]
    args = [ag, bg, vg, edge_attr, c_w, c_b, gamma_e, beta_e]
    aliases = {}
    if e_buf is not None:
        in_specs.append(pl.BlockSpec(memory_space=pl.ANY))
        args.append(e_buf)
        aliases = {8: 0}
    return pl.pallas_call(
        _edge_body,
        grid=grid,
        in_specs=in_specs,
        out_specs=[
            pl.BlockSpec((block_rows, d), lambda i: (ob + i, 0)),
            pl.BlockSpec((2, block_rows, dh), lambda i: (0, i, 0)),
        ],
        out_shape=[
            jax.ShapeDtypeStruct((e_total, d), jnp.float32),
            jax.ShapeDtypeStruct((2, es, dh), jnp.float32),
        ],
        input_output_aliases=aliases,
    )(*args)


def _scatter_body(msg_hbm, dst_hbm, zero_hbm, agg_hbm,
                  idx_d, msg, agg_sp, sems, chunk, n_iter, dh, nbuf):
    c = lax.axis_index("c")
    s = lax.axis_index("s")
    n = agg_sp.shape[0]
    e2 = msg_hbm.shape[0]
    e = e2 // 2
    per_tile = chunk * n_iter
    base0 = s * per_tile
    ce0 = c * e

    def start_loads(i, b):
        pltpu.async_copy(dst_hbm.at[pl.ds(base0 + i * chunk, chunk)],
                         idx_d.at[b], sems.at[0, b])
        pltpu.async_copy(msg_hbm.at[pl.ds(ce0 + base0 + i * chunk, chunk)],
                         msg.at[b], sems.at[1, b])

    def wait_loads(b):
        pltpu.make_async_copy(dst_hbm.at[pl.ds(0, chunk)], idx_d.at[b],
                              sems.at[0, b]).wait()
        pltpu.make_async_copy(msg_hbm.at[pl.ds(0, chunk)], msg.at[b],
                              sems.at[1, b]).wait()

    def start_scadd(b):
        pltpu.async_copy(msg.at[b], agg_sp.at[idx_d.at[b]], sems.at[2, b],
                         add=True)

    def wait_scadd(b):
        pltpu.make_async_copy(msg.at[b], agg_sp.at[idx_d.at[b]],
                              sems.at[2, b]).wait()

    def step(i, b, has_idxn):
        wait_loads(b)
        start_scadd(b)
        if has_idxn:
            wait_scadd(b)
            start_loads(i + nbuf, b)

    for i in range(min(nbuf, n_iter)):
        start_loads(i, i % nbuf)

    rpt = -(-n // NS) // 8 * 8
    lastr = n - (NS - 1) * rpt

    @pl.when(s < NS - 1)
    def _():
        pltpu.sync_copy(zero_hbm.at[pl.ds(s * rpt, rpt)],
                        agg_sp.at[pl.ds(s * rpt, rpt)])

    @pl.when(s == NS - 1)
    def _():
        pltpu.sync_copy(zero_hbm.at[pl.ds((NS - 1) * rpt, lastr)],
                        agg_sp.at[pl.ds((NS - 1) * rpt, lastr)])

    plsc.subcore_barrier()

    for i in range(min(nbuf, n_iter)):
        step(i, i % nbuf, has_idxn=i + nbuf < n_iter)

    lo = nbuf
    hi = max(((n_iter - 1 - nbuf) // nbuf) * nbuf, lo)
    if hi > lo:
        def grp(m, _):
            i0 = lo + m * nbuf
            for k in range(nbuf):
                step(i0 + k, k, has_idxn=True)
            return 0

        lax.fori_loop(0, (hi - lo) // nbuf, grp, 0)

    for i in range(hi, n_iter):
        if i < nbuf:
            continue
        step(i, i % nbuf, has_idxn=i + nbuf < n_iter)

    for i in range(max(n_iter - nbuf, 0), n_iter):
        wait_scadd(i % nbuf)

    plsc.subcore_barrier()

    @pl.when(s < NS - 1)
    def _():
        pltpu.sync_copy(agg_sp.at[pl.ds(s * rpt, rpt)],
                        agg_hbm.at[pl.ds(c * n + s * rpt, rpt)])

    @pl.when(s == NS - 1)
    def _():
        pltpu.sync_copy(agg_sp.at[pl.ds((NS - 1) * rpt, lastr)],
                        agg_hbm.at[pl.ds(c * n + (NS - 1) * rpt, lastr)])


def _sc_scatter(msg2, dst, zeros_half):
    e2, dh = msg2.shape
    e = e2 // 2
    n = zeros_half.shape[0]
    chunk = 80
    nbuf = 3
    n_iter = e // (NS * chunk)
    assert e == NS * chunk * n_iter
    k = functools.partial(
        pl.kernel,
        out_type=jax.ShapeDtypeStruct((2 * n, dh), jnp.float32),
        mesh=_sc_mesh(),
        scratch_types=[
            pltpu.VMEM((nbuf, chunk), jnp.int32),
            pltpu.VMEM((nbuf, chunk, dh), jnp.float32),
            pltpu.VMEM_SHARED((n, dh), jnp.float32),
            pltpu.SemaphoreType.DMA((3, nbuf)),
        ],
    )(functools.partial(_scatter_body, chunk=chunk, n_iter=n_iter, dh=dh,
                        nbuf=nbuf))
    return k(msg2, dst, zeros_half)


def _epi_body(h_ref, uh_ref, gh_ref, bh_ref, *rest):
    agg_refs, o_ref = rest[:-1], rest[-1]
    agg = sum(jnp.concatenate([a[0], a[1]], axis=1) for a in agg_refs)
    x = uh_ref[...] + agg
    out = x * jax.nn.sigmoid(x)
    y = h_ref[...] + out
    mu = jnp.mean(y, axis=-1, keepdims=True)
    var = jnp.mean((y - mu) ** 2, axis=-1, keepdims=True)
    o_ref[...] = (y - mu) * lax.rsqrt(var + 1e-5) * gh_ref[...] + bh_ref[...]


def _epilogue(h, uh, agg2s, gamma_h, beta_h, block_rows):
    n, d = h.shape
    dh = d // 2
    grid = (n // block_rows,)
    return pl.pallas_call(
        _epi_body,
        grid=grid,
        in_specs=[
            pl.BlockSpec((block_rows, d), lambda i: (i, 0)),
            pl.BlockSpec((block_rows, d), lambda i: (i, 0)),
            pl.BlockSpec((1, d), lambda i: (0, 0)),
            pl.BlockSpec((1, d), lambda i: (0, 0)),
        ] + [
            pl.BlockSpec((2, block_rows, dh), lambda i: (0, i, 0))
            for _ in agg2s
        ],
        out_specs=pl.BlockSpec((block_rows, d), lambda i: (i, 0)),
        out_shape=jax.ShapeDtypeStruct((n, d), jnp.float32),
    )(h, uh, gamma_h, beta_h, *agg2s)


def kernel(h, edge_index, edge_attr, U_w, U_b, V_w, V_b, A_w, A_b, B_w, B_b,
           C_w, C_b, gamma_h, beta_h, gamma_e, beta_e):
    n, d = h.shape
    dh = d // 2
    src = edge_index[0]
    dst = edge_index[1]

    w_all = jnp.concatenate([U_w, A_w, B_w, V_w], axis=1)
    b_all = jnp.concatenate([U_b, A_b, B_b, V_b])[None, :]
    uh, aa, ba, va2 = _node_matmul(h, w_all, b_all, block_rows=2000)

    e = src.shape[0]
    unit = NW * 40
    n_units = e // unit
    assert e == unit * n_units
    nslab = 3
    per = n_units // nslab
    sizes = [(per + (1 if i < n_units % nslab else 0)) * unit
             for i in range(nslab)]
    starts = [sum(sizes[:i]) for i in range(nslab)]

    zeros_half = jnp.zeros((n, dh), jnp.float32)

    e_buf = jnp.zeros((0,))
    agg2s = []
    e_total = e
    for si, (st, sz) in enumerate(zip(starts, sizes)):
        dst_s = lax.slice(dst, (st,), (st + sz,))
        src_s = lax.slice(src, (st,), (st + sz,))
        ea_s = lax.slice(edge_attr, (st, 0), (st + sz, edge_attr.shape[1]))
        ag, bg, vg = _sc_gather(aa, ba, va2, dst_s, src_s)
        e_init = None if si == 0 else e_buf
        e_buf, msg2 = _edge_math(ag, bg, vg, ea_s, C_w, C_b[None, :],
                                 gamma_e[None, :], beta_e[None, :],
                                 block_rows=1280, e_buf=e_init,
                                 e_total=e_total, row_off=st)
        agg2s.append(_sc_scatter(msg2.reshape(-1, dh), dst_s, zeros_half))

    e_out = e_buf
    h_out = _epilogue(h, uh, [a.reshape(2, n, dh) for a in agg2s],
                      gamma_h[None, :], beta_h[None, :], block_rows=1000)
    return (h_out, e_out)


# speedup vs baseline: 1.0216x; 1.0009x over previous
import functools

import jax
import jax.numpy as jnp
from jax import lax
from jax.experimental import pallas as pl
from jax.experimental.pallas import tpu as pltpu
from jax.experimental.pallas import tpu_sc as plsc

NC = 2
NS = 16
NW = NC * NS
L = 16


def _sc_mesh():
    return plsc.VectorSubcoreMesh(core_axis_name="c", subcore_axis_name="s",
                                  num_cores=NC, num_subcores=NS)


def _pack_bf16_pairs(x):
    k = x.shape[1] // 2
    lo = lax.bitcast_convert_type(x[:, :k], jnp.uint32)
    hi = lax.bitcast_convert_type(x[:, k:], jnp.uint32)
    lo16 = lax.shift_right_logical(lo + jnp.uint32(0x8000), jnp.uint32(16))
    hi16 = (hi + jnp.uint32(0x8000)) & jnp.uint32(0xFFFF0000)
    return lax.bitcast_convert_type(hi16 | lo16, jnp.int32)


def _unpack_bf16_pairs(p):
    u = lax.bitcast_convert_type(p, jnp.uint32)
    lo = lax.bitcast_convert_type(lax.shift_left(u, jnp.uint32(16)),
                                  jnp.float32)
    hi = lax.bitcast_convert_type(u & jnp.uint32(0xFFFF0000), jnp.float32)
    return jnp.concatenate([lo, hi], axis=1)


def _mm_body(x_ref, w_ref, b_ref, uh_ref, aa_ref, ba_ref, va_ref):
    d = x_ref.shape[1]
    o = (
        jnp.dot(x_ref[...], w_ref[...], preferred_element_type=jnp.float32)
        + b_ref[...]
    )
    uh_ref[...] = o[:, :d]
    aa_ref[...] = _pack_bf16_pairs(o[:, d:2 * d])
    ba_ref[...] = _pack_bf16_pairs(o[:, 2 * d:3 * d])
    va_ref[...] = _pack_bf16_pairs(o[:, 3 * d:])


def _node_matmul(h, w_all, b_all, block_rows):
    n, d = h.shape
    dout = w_all.shape[1]
    dh = d // 2
    grid = (n // block_rows,)
    nd = jax.ShapeDtypeStruct((n, d), jnp.float32)
    nb = jax.ShapeDtypeStruct((n, d // 2), jnp.int32)
    return pl.pallas_call(
        _mm_body,
        grid=grid,
        in_specs=[
            pl.BlockSpec((block_rows, d), lambda i: (i, 0)),
            pl.BlockSpec((d, dout), lambda i: (0, 0)),
            pl.BlockSpec((1, dout), lambda i: (0, 0)),
        ],
        out_specs=[
            pl.BlockSpec((block_rows, d), lambda i: (i, 0)),
            pl.BlockSpec((block_rows, d // 2), lambda i: (i, 0)),
            pl.BlockSpec((block_rows, d // 2), lambda i: (i, 0)),
            pl.BlockSpec((block_rows, d // 2), lambda i: (i, 0)),
        ],
        out_shape=[nd, nb, nb, nb],
    )(h, w_all, b_all)


_NBUF = 3


def _gather_body(aa_hbm, ba_hbm, va_hbm, dst_hbm, src_hbm,
                 ag_hbm, bg_hbm, vg_hbm,
                 idx_d, idx_s, rows_a, rows_b, rows_v, sems, chunk, n_iter):
    wid = lax.axis_index("c") * NS + lax.axis_index("s")
    per_tile = chunk * n_iter
    base0 = wid * per_tile

    def start_idx(i, b):
        pltpu.async_copy(dst_hbm.at[pl.ds(base0 + i * chunk, chunk)],
                         idx_d.at[b], sems.at[0, b])
        pltpu.async_copy(src_hbm.at[pl.ds(base0 + i * chunk, chunk)],
                         idx_s.at[b], sems.at[1, b])

    def wait_idx(b):
        pltpu.make_async_copy(dst_hbm.at[pl.ds(0, chunk)], idx_d.at[b],
                              sems.at[0, b]).wait()
        pltpu.make_async_copy(src_hbm.at[pl.ds(0, chunk)], idx_s.at[b],
                              sems.at[1, b]).wait()

    def start_gather(b):
        pltpu.async_copy(aa_hbm.at[idx_d.at[b]], rows_a.at[b], sems.at[2, b])
        pltpu.async_copy(ba_hbm.at[idx_s.at[b]], rows_b.at[b], sems.at[3, b])
        pltpu.async_copy(va_hbm.at[idx_s.at[b]], rows_v.at[b], sems.at[4, b])

    def wait_gather(b):
        pltpu.make_async_copy(aa_hbm.at[idx_d.at[b]], rows_a.at[b],
                              sems.at[2, b]).wait()
        pltpu.make_async_copy(ba_hbm.at[idx_s.at[b]], rows_b.at[b],
                              sems.at[3, b]).wait()
        pltpu.make_async_copy(va_hbm.at[idx_s.at[b]], rows_v.at[b],
                              sems.at[4, b]).wait()

    def start_wb(i, b):
        pltpu.async_copy(rows_a.at[b],
                         ag_hbm.at[pl.ds(base0 + i * chunk, chunk)],
                         sems.at[5, b])
        pltpu.async_copy(rows_b.at[b],
                         bg_hbm.at[pl.ds(base0 + i * chunk, chunk)],
                         sems.at[6, b])
        pltpu.async_copy(rows_v.at[b],
                         vg_hbm.at[pl.ds(base0 + i * chunk, chunk)],
                         sems.at[7, b])

    def wait_wb(b):
        pltpu.make_async_copy(rows_a.at[b], ag_hbm.at[pl.ds(0, chunk)],
                              sems.at[5, b]).wait()
        pltpu.make_async_copy(rows_b.at[b], bg_hbm.at[pl.ds(0, chunk)],
                              sems.at[6, b]).wait()
        pltpu.make_async_copy(rows_v.at[b], vg_hbm.at[pl.ds(0, chunk)],
                              sems.at[7, b]).wait()

    def step(i, b, has_next, do_wbwait, has_idx4):
        wait_gather(b)
        start_wb(i, b)
        if has_next:
            b1 = (b + 1) % _NBUF
            wait_idx(b1)
            if do_wbwait:
                wait_wb(b1)
            start_gather(b1)
        if has_idx4:
            start_idx(i + _NBUF, b)

    for i in range(min(_NBUF, n_iter)):
        start_idx(i, i % _NBUF)
    wait_idx(0)
    start_gather(0)

    for i in range(min(_NBUF, n_iter)):
        step(i, i % _NBUF, has_next=i + 1 < n_iter,
             do_wbwait=i >= _NBUF - 1, has_idx4=i + _NBUF < n_iter)

    lo = _NBUF
    hi = max(((n_iter - 1 - _NBUF) // _NBUF) * _NBUF, lo)
    if hi > lo:
        def grp(m, _):
            i0 = lo + m * _NBUF
            for k in range(_NBUF):
                step(i0 + k, k, has_next=True, do_wbwait=True, has_idx4=True)
            return 0

        lax.fori_loop(0, (hi - lo) // _NBUF, grp, 0)

    for i in range(hi, n_iter):
        if i < _NBUF:
            continue
        step(i, i % _NBUF, has_next=i + 1 < n_iter, do_wbwait=True,
             has_idx4=i + _NBUF < n_iter)

    for i in range(max(n_iter - _NBUF, 0), n_iter):
        wait_wb(i % _NBUF)


def _sc_gather(aa, ba, va, dst, src):
    n, d = aa.shape
    e = dst.shape[0]
    chunk = 40
    n_iter = e // (NW * chunk)
    assert e == NW * chunk * n_iter
    k = functools.partial(
        pl.kernel,
        out_type=(jax.ShapeDtypeStruct((e, d), jnp.int32),
                  jax.ShapeDtypeStruct((e, d), jnp.int32),
                  jax.ShapeDtypeStruct((e, d), jnp.int32)),
        mesh=_sc_mesh(),
        scratch_types=[
            pltpu.VMEM((_NBUF, chunk), jnp.int32),
            pltpu.VMEM((_NBUF, chunk), jnp.int32),
            pltpu.VMEM((_NBUF, chunk, d), jnp.int32),
            pltpu.VMEM((_NBUF, chunk, d), jnp.int32),
            pltpu.VMEM((_NBUF, chunk, d), jnp.int32),
            pltpu.SemaphoreType.DMA((8, _NBUF)),
        ],
    )(functools.partial(_gather_body, chunk=chunk, n_iter=n_iter))
    return k(aa, ba, va, dst, src)


def _edge_body(ag_ref, bg_ref, vg_ref, ea_ref, cw_ref, cb_ref, ge_ref,
               be_ref, *rest):
    if len(rest) == 3:
        _, eout_ref, msg_ref = rest
    else:
        eout_ref, msg_ref = rest
    d = 2 * ag_ref.shape[1]
    dh = d // 2
    ce = (
        jnp.dot(ea_ref[...], cw_ref[...], preferred_element_type=jnp.float32)
        + cb_ref[...]
    )
    e = _unpack_bf16_pairs(ag_ref[...]) + _unpack_bf16_pairs(bg_ref[...]) + ce
    gates = jax.nn.sigmoid(e)
    msg = gates * _unpack_bf16_pairs(vg_ref[...])
    msg_ref[0] = msg[:, :dh]
    msg_ref[1] = msg[:, dh:]
    se = e * gates
    mu = jnp.mean(se, axis=-1, keepdims=True)
    var = jnp.mean((se - mu) ** 2, axis=-1, keepdims=True)
    eout_ref[...] = (se - mu) * lax.rsqrt(var + 1e-5) * ge_ref[...] + be_ref[...]


def _edge_math(ag, bg, vg, edge_attr, c_w, c_b, gamma_e, beta_e,
               block_rows, e_buf, e_total, row_off):
    es, dp = ag.shape
    d = 2 * dp
    ed = edge_attr.shape[1]
    dh = d // 2
    grid = (es // block_rows,)
    ob = row_off // block_rows
    in_specs = [
        pl.BlockSpec((block_rows, dp), lambda i: (i, 0)),
        pl.BlockSpec((block_rows, dp), lambda i: (i, 0)),
        pl.BlockSpec((block_rows, dp), lambda i: (i, 0)),
        pl.BlockSpec((block_rows, ed), lambda i: (i, 0)),
        pl.BlockSpec((ed, d), lambda i: (0, 0)),
        pl.BlockSpec((1, d), lambda i: (0, 0)),
        pl.BlockSpec((1, d), lambda i: (0, 0)),
        pl.BlockSpec((1, d), lambda i: (0, 0)),
    ]
    args = [ag, bg, vg, edge_attr, c_w, c_b, gamma_e, beta_e]
    aliases = {}
    if e_buf is not None:
        in_specs.append(pl.BlockSpec(memory_space=pl.ANY))
        args.append(e_buf)
        aliases = {8: 0}
    return pl.pallas_call(
        _edge_body,
        grid=grid,
        in_specs=in_specs,
        out_specs=[
            pl.BlockSpec((block_rows, d), lambda i: (ob + i, 0)),
            pl.BlockSpec((2, block_rows, dh), lambda i: (0, i, 0)),
        ],
        out_shape=[
            jax.ShapeDtypeStruct((e_total, d), jnp.float32),
            jax.ShapeDtypeStruct((2, es, dh), jnp.float32),
        ],
        input_output_aliases=aliases,
    )(*args)


def _scatter_body(msg_hbm, dst_hbm, zero_hbm, agg_hbm,
                  idx_d, msg, agg_sp, sems, chunk, n_iter, dh, nbuf):
    c = lax.axis_index("c")
    s = lax.axis_index("s")
    n = agg_sp.shape[0]
    e2 = msg_hbm.shape[0]
    e = e2 // 2
    per_tile = chunk * n_iter
    base0 = s * per_tile
    ce0 = c * e

    def start_loads(i, b):
        pltpu.async_copy(dst_hbm.at[pl.ds(base0 + i * chunk, chunk)],
                         idx_d.at[b], sems.at[0, b])
        pltpu.async_copy(msg_hbm.at[pl.ds(ce0 + base0 + i * chunk, chunk)],
                         msg.at[b], sems.at[1, b])

    def wait_loads(b):
        pltpu.make_async_copy(dst_hbm.at[pl.ds(0, chunk)], idx_d.at[b],
                              sems.at[0, b]).wait()
        pltpu.make_async_copy(msg_hbm.at[pl.ds(0, chunk)], msg.at[b],
                              sems.at[1, b]).wait()

    def start_scadd(b):
        pltpu.async_copy(msg.at[b], agg_sp.at[idx_d.at[b]], sems.at[2, b],
                         add=True)

    def wait_scadd(b):
        pltpu.make_async_copy(msg.at[b], agg_sp.at[idx_d.at[b]],
                              sems.at[2, b]).wait()

    def step(i, b, has_idxn):
        wait_loads(b)
        start_scadd(b)
        if has_idxn:
            wait_scadd(b)
            start_loads(i + nbuf, b)

    for i in range(min(nbuf, n_iter)):
        start_loads(i, i % nbuf)

    rpt = -(-n // NS) // 8 * 8
    lastr = n - (NS - 1) * rpt

    @pl.when(s < NS - 1)
    def _():
        pltpu.sync_copy(zero_hbm.at[pl.ds(s * rpt, rpt)],
                        agg_sp.at[pl.ds(s * rpt, rpt)])

    @pl.when(s == NS - 1)
    def _():
        pltpu.sync_copy(zero_hbm.at[pl.ds((NS - 1) * rpt, lastr)],
                        agg_sp.at[pl.ds((NS - 1) * rpt, lastr)])

    plsc.subcore_barrier()

    for i in range(min(nbuf, n_iter)):
        step(i, i % nbuf, has_idxn=i + nbuf < n_iter)

    lo = nbuf
    hi = max(((n_iter - 1 - nbuf) // nbuf) * nbuf, lo)
    if hi > lo:
        def grp(m, _):
            i0 = lo + m * nbuf
            for k in range(nbuf):
                step(i0 + k, k, has_idxn=True)
            return 0

        lax.fori_loop(0, (hi - lo) // nbuf, grp, 0)

    for i in range(hi, n_iter):
        if i < nbuf:
            continue
        step(i, i % nbuf, has_idxn=i + nbuf < n_iter)

    for i in range(max(n_iter - nbuf, 0), n_iter):
        wait_scadd(i % nbuf)

    plsc.subcore_barrier()

    @pl.when(s < NS - 1)
    def _():
        pltpu.sync_copy(agg_sp.at[pl.ds(s * rpt, rpt)],
                        agg_hbm.at[pl.ds(c * n + s * rpt, rpt)])

    @pl.when(s == NS - 1)
    def _():
        pltpu.sync_copy(agg_sp.at[pl.ds((NS - 1) * rpt, lastr)],
                        agg_hbm.at[pl.ds(c * n + (NS - 1) * rpt, lastr)])


def _sc_scatter(msg2, dst, zeros_half):
    e2, dh = msg2.shape
    e = e2 // 2
    n = zeros_half.shape[0]
    chunk = 80
    nbuf = 3
    n_iter = e // (NS * chunk)
    assert e == NS * chunk * n_iter
    k = functools.partial(
        pl.kernel,
        out_type=jax.ShapeDtypeStruct((2 * n, dh), jnp.float32),
        mesh=_sc_mesh(),
        scratch_types=[
            pltpu.VMEM((nbuf, chunk), jnp.int32),
            pltpu.VMEM((nbuf, chunk, dh), jnp.float32),
            pltpu.VMEM_SHARED((n, dh), jnp.float32),
            pltpu.SemaphoreType.DMA((3, nbuf)),
        ],
    )(functools.partial(_scatter_body, chunk=chunk, n_iter=n_iter, dh=dh,
                        nbuf=nbuf))
    return k(msg2, dst, zeros_half)


def _epi_body(h_ref, uh_ref, gh_ref, bh_ref, *rest):
    agg_refs, o_ref = rest[:-1], rest[-1]
    agg = sum(jnp.concatenate([a[0], a[1]], axis=1) for a in agg_refs)
    x = uh_ref[...] + agg
    out = x * jax.nn.sigmoid(x)
    y = h_ref[...] + out
    mu = jnp.mean(y, axis=-1, keepdims=True)
    var = jnp.mean((y - mu) ** 2, axis=-1, keepdims=True)
    o_ref[...] = (y - mu) * lax.rsqrt(var + 1e-5) * gh_ref[...] + bh_ref[...]


def _epilogue(h, uh, agg2s, gamma_h, beta_h, block_rows):
    n, d = h.shape
    dh = d // 2
    grid = (n // block_rows,)
    return pl.pallas_call(
        _epi_body,
        grid=grid,
        in_specs=[
            pl.BlockSpec((block_rows, d), lambda i: (i, 0)),
            pl.BlockSpec((block_rows, d), lambda i: (i, 0)),
            pl.BlockSpec((1, d), lambda i: (0, 0)),
            pl.BlockSpec((1, d), lambda i: (0, 0)),
        ] + [
            pl.BlockSpec((2, block_rows, dh), lambda i: (0, i, 0))
            for _ in agg2s
        ],
        out_specs=pl.BlockSpec((block_rows, d), lambda i: (i, 0)),
        out_shape=jax.ShapeDtypeStruct((n, d), jnp.float32),
    )(h, uh, gamma_h, beta_h, *agg2s)


def kernel(h, edge_index, edge_attr, U_w, U_b, V_w, V_b, A_w, A_b, B_w, B_b,
           C_w, C_b, gamma_h, beta_h, gamma_e, beta_e):
    n, d = h.shape
    dh = d // 2
    src = edge_index[0]
    dst = edge_index[1]

    w_all = jnp.concatenate([U_w, A_w, B_w, V_w], axis=1)
    b_all = jnp.concatenate([U_b, A_b, B_b, V_b])[None, :]
    uh, aa, ba, va2 = _node_matmul(h, w_all, b_all, block_rows=2000)

    e = src.shape[0]
    unit = NW * 40
    n_units = e // unit
    assert e == unit * n_units
    nslab = 3
    per = n_units // nslab
    sizes = [(per + (1 if i < n_units % nslab else 0)) * unit
             for i in range(nslab)]
    starts = [sum(sizes[:i]) for i in range(nslab)]

    zeros_half = jnp.zeros((n, dh), jnp.float32)

    e_buf = None
    agg2s = []
    e_total = e
    for si, (st, sz) in enumerate(zip(starts, sizes)):
        dst_s = lax.slice(dst, (st,), (st + sz,))
        src_s = lax.slice(src, (st,), (st + sz,))
        ea_s = lax.slice(edge_attr, (st, 0), (st + sz, edge_attr.shape[1]))
        ag, bg, vg = _sc_gather(aa, ba, va2, dst_s, src_s)
        e_init = None if si == 0 else e_buf
        e_buf, msg2 = _edge_math(ag, bg, vg, ea_s, C_w, C_b[None, :],
                                 gamma_e[None, :], beta_e[None, :],
                                 block_rows=1280, e_buf=e_init,
                                 e_total=e_total, row_off=st)
        agg2s.append(_sc_scatter(msg2.reshape(-1, dh), dst_s, zeros_half))

    e_out = e_buf
    h_out = _epilogue(h, uh, [a.reshape(2, n, dh) for a in agg2s],
                      gamma_h[None, :], beta_h[None, :], block_rows=1000)
    return (h_out, e_out)
